# baseline (speedup 1.0000x reference)
import functools

import jax
import jax.numpy as jnp
from jax import lax
from jax.experimental import pallas as pl
from jax.experimental.pallas import tpu as pltpu
from jax.experimental.pallas import tpu_sc as plsc

N = 2048
D = 1024
E = 8
K = 2
EMB = 1024
H1 = 2048
H2 = 1024
DO = 1024

RBLK = 256
BLK = 128
MAXPAD = 5120
NBLK = MAXPAD // BLK

NC, NS = 2, 16
NW = NC * NS


def _router_body(x_ref, wr_ref, br_ref, emb_ref, i0_ref, i1_ref, g0_ref,
                 g1_ref):
    x = x_ref[...]
    q = jnp.dot(x.astype(jnp.bfloat16), wr_ref[...].astype(jnp.bfloat16),
                preferred_element_type=jnp.float32)
    q = q + br_ref[...]
    cols = []
    for e in range(E):
        de = q - emb_ref[e, :][None, :]
        cols.append(-jnp.sum(de * de, axis=1, keepdims=True))
    scores = jnp.concatenate(cols, axis=1)
    iota = lax.broadcasted_iota(jnp.int32, scores.shape, 1)
    neginf = jnp.float32(-jnp.inf)
    m1 = jnp.max(scores, axis=1, keepdims=True)
    a1 = jnp.min(jnp.where(scores == m1, iota, E), axis=1, keepdims=True)
    masked = jnp.where(iota == a1, neginf, scores)
    m2 = jnp.max(masked, axis=1, keepdims=True)
    a2 = jnp.min(jnp.where(masked == m2, iota, E), axis=1, keepdims=True)
    g = 1.0 / (1.0 + jnp.exp(m2 - m1))
    i0_ref[...] = a1
    i1_ref[...] = a2
    g0_ref[...] = g
    g1_ref[...] = 1.0 - g


def _run_router(x, wr, br, emb):
    out_shapes = (
        jax.ShapeDtypeStruct((N, 1), jnp.int32),
        jax.ShapeDtypeStruct((N, 1), jnp.int32),
        jax.ShapeDtypeStruct((N, 1), jnp.float32),
        jax.ShapeDtypeStruct((N, 1), jnp.float32),
    )
    ospec = pl.BlockSpec((RBLK, 1), lambda i: (i, 0))
    return pl.pallas_call(
        _router_body,
        grid=(N // RBLK,),
        in_specs=[
            pl.BlockSpec((RBLK, D), lambda i: (i, 0)),
            pl.BlockSpec((D, EMB), lambda i: (0, 0)),
            pl.BlockSpec((1, EMB), lambda i: (0, 0)),
            pl.BlockSpec((E, EMB), lambda i: (0, 0)),
        ],
        out_specs=(ospec, ospec, ospec, ospec),
        out_shape=out_shapes,
    )(x, wr, br.reshape(1, EMB), emb)


def _route_metadata(i0, i1, g0, g1):
    flat_e = jnp.concatenate([i0, i1], axis=1).reshape(N * K)
    flat_g = jnp.concatenate([g0, g1], axis=1).reshape(N * K)
    oh = (flat_e[:, None] == jnp.arange(E)[None, :]).astype(jnp.int32)
    cum = jnp.cumsum(oh, axis=0)
    counts = cum[-1]
    rank = jnp.take_along_axis(cum, flat_e[:, None], axis=1).reshape(-1) - 1
    padded = ((counts + BLK - 1) // BLK) * BLK
    cum_pad = jnp.cumsum(padded)
    pad_start = cum_pad - padded
    padded_pos = (pad_start[flat_e] + rank).astype(jnp.int32)
    tok = jnp.arange(N * K, dtype=jnp.int32) // K
    row_token = jnp.zeros((MAXPAD,), jnp.int32).at[padded_pos].set(tok)
    row_gate = jnp.zeros((MAXPAD,), jnp.float32).at[padded_pos].set(flat_g)
    blk_e = jnp.searchsorted(cum_pad, jnp.arange(NBLK) * BLK, side="right")
    blk_e = jnp.minimum(blk_e, E - 1).astype(jnp.int32)
    pp = padded_pos.reshape(N, K)
    return row_token, row_gate, blk_e, pp[:, 0], pp[:, 1]


def _sc_gather(x, row_token):
    rows_per_w = MAXPAD // NW
    ch = 32
    nch = rows_per_w // ch
    nb = 3
    mesh = plsc.VectorSubcoreMesh(core_axis_name="c", subcore_axis_name="s",
                                  num_cores=NC, num_subcores=NS)

    @functools.partial(
        pl.kernel, mesh=mesh,
        out_type=jax.ShapeDtypeStruct((MAXPAD, D), jnp.float32),
        scratch_types=[
            pltpu.VMEM((rows_per_w,), jnp.int32),
            [pltpu.VMEM((ch, D), jnp.float32)] * nb,
            [pltpu.SemaphoreType.DMA] * nb,
            [pltpu.SemaphoreType.DMA] * nb,
        ],
    )
    def k(x_hbm, idx_hbm, out_hbm, idx_v, bufs, gsems, ssems):
        wid = lax.axis_index("s") * NC + lax.axis_index("c")
        base = wid * rows_per_w
        pltpu.sync_copy(idx_hbm.at[pl.ds(base, rows_per_w)], idx_v)

        def fire(c):
            return pltpu.async_copy(x_hbm.at[idx_v.at[pl.ds(c * ch, ch)]],
                                    bufs[c % nb], gsems[c % nb])

        gd = {c: fire(c) for c in range(nb - 1)}
        stores = {}
        for c in range(nch):
            b = c % nb
            f = c + nb - 1
            if f < nch:
                fb = f % nb
                if fb in stores:
                    stores[fb].wait()
                gd[f] = fire(f)
            gd[c].wait()
            stores[b] = pltpu.async_copy(
                bufs[b], out_hbm.at[pl.ds(base + c * ch, ch)], ssems[b])
        for b in set(c % nb for c in range(max(0, nch - nb), nch)):
            stores[b].wait()

    return k(x, row_token)


def _mlp_body(be_ref, xs_ref, gate_ref, w1_ref, b1_ref, w2_ref, b2_ref,
              w3_ref, b3_ref, out_ref):
    bf = jnp.bfloat16
    x = xs_ref[...]
    h = jnp.dot(x.astype(bf), w1_ref[0].astype(bf),
                preferred_element_type=jnp.float32)
    h = jnp.maximum(h + b1_ref[0], 0.0)
    h = jnp.dot(h.astype(bf), w2_ref[0].astype(bf),
                preferred_element_type=jnp.float32)
    h = jnp.maximum(h + b2_ref[0], 0.0)
    y = jnp.dot(h.astype(bf), w3_ref[0].astype(bf),
                preferred_element_type=jnp.float32)
    y = y + b3_ref[0]
    out_ref[...] = y * gate_ref[...]


def _run_mlp(xs, row_gate, blk_e, w1, b1, w2, b2, w3, b3):
    grid_spec = pltpu.PrefetchScalarGridSpec(
        num_scalar_prefetch=1,
        grid=(NBLK,),
        in_specs=[
            pl.BlockSpec((BLK, D), lambda i, be: (i, 0)),
            pl.BlockSpec((BLK, 1), lambda i, be: (i, 0)),
            pl.BlockSpec((1, D, H1), lambda i, be: (be[i], 0, 0)),
            pl.BlockSpec((1, 1, H1), lambda i, be: (be[i], 0, 0)),
            pl.BlockSpec((1, H1, H2), lambda i, be: (be[i], 0, 0)),
            pl.BlockSpec((1, 1, H2), lambda i, be: (be[i], 0, 0)),
            pl.BlockSpec((1, H2, DO), lambda i, be: (be[i], 0, 0)),
            pl.BlockSpec((1, 1, DO), lambda i, be: (be[i], 0, 0)),
        ],
        out_specs=pl.BlockSpec((BLK, DO), lambda i, be: (i, 0)),
    )
    return pl.pallas_call(
        _mlp_body,
        grid_spec=grid_spec,
        out_shape=jax.ShapeDtypeStruct((MAXPAD, DO), jnp.float32),
    )(blk_e, xs, row_gate.reshape(MAXPAD, 1),
      w1, b1.reshape(E, 1, H1), w2, b2.reshape(E, 1, H2),
      w3, b3.reshape(E, 1, DO))


def _sc_combine(ys, pos0, pos1):
    tok_per_w = N // NW
    ch = 16
    mesh = plsc.VectorSubcoreMesh(core_axis_name="c", subcore_axis_name="s",
                                  num_cores=NC, num_subcores=NS)

    @functools.partial(
        pl.kernel, mesh=mesh,
        out_type=jax.ShapeDtypeStruct((N, DO), jnp.float32),
        scratch_types=[
            pltpu.VMEM((tok_per_w,), jnp.int32),
            pltpu.VMEM((tok_per_w,), jnp.int32),
            pltpu.VMEM((ch, DO), jnp.float32),
            pltpu.VMEM((ch, DO), jnp.float32),
            pltpu.VMEM((ch, DO), jnp.float32),
            pltpu.VMEM((ch, DO), jnp.float32),
            pltpu.SemaphoreType.DMA,
            pltpu.SemaphoreType.DMA,
            pltpu.SemaphoreType.DMA,
            pltpu.SemaphoreType.DMA,
            pltpu.SemaphoreType.DMA,
            pltpu.SemaphoreType.DMA,
        ],
    )
    def k(ys_hbm, p0_hbm, p1_hbm, out_hbm, i0_v, i1_v, a0, a1, b0, b1,
          ga0, ga1, gb0, gb1, s0, s1):
        wid = lax.axis_index("s") * NC + lax.axis_index("c")
        base = wid * tok_per_w
        pltpu.sync_copy(p0_hbm.at[pl.ds(base, tok_per_w)], i0_v)
        pltpu.sync_copy(p1_hbm.at[pl.ds(base, tok_per_w)], i1_v)
        abufs, bbufs = (a0, a1), (b0, b1)
        gasems, gbsems, ssems = (ga0, ga1), (gb0, gb1), (s0, s1)
        nch = tok_per_w // ch
        d0 = pltpu.async_copy(ys_hbm.at[i0_v.at[pl.ds(0, ch)]], a0, ga0)
        d1 = pltpu.async_copy(ys_hbm.at[i1_v.at[pl.ds(0, ch)]], b0, gb0)
        stores = [None, None]
        for c in range(nch):
            b = c & 1
            nd0 = nd1 = None
            if c + 1 < nch:
                ob = (c + 1) & 1
                if stores[ob] is not None:
                    stores[ob].wait()
                sl = pl.ds((c + 1) * ch, ch)
                nd0 = pltpu.async_copy(ys_hbm.at[i0_v.at[sl]], abufs[ob],
                                       gasems[ob])
                nd1 = pltpu.async_copy(ys_hbm.at[i1_v.at[sl]], bbufs[ob],
                                       gbsems[ob])
            d0.wait()
            d1.wait()
            a_ref, b_ref = abufs[b], bbufs[b]

            def row_body(r, _, a_ref=a_ref, b_ref=b_ref):
                for cc in range(DO // 16):
                    s = pl.ds(cc * 16, 16)
                    a_ref[r, s] = a_ref[r, s] + b_ref[r, s]
                return 0

            lax.fori_loop(0, ch, row_body, 0)
            stores[b] = pltpu.async_copy(a_ref,
                                         out_hbm.at[pl.ds(base + c * ch, ch)],
                                         ssems[b])
            d0, d1 = nd0, nd1
        stores[0].wait()
        stores[1].wait()

    return k(ys, pos0, pos1)


def kernel(inputs, Wr, br, expert_embeddings, W1, b1, W2, b2, W3, b3):
    i0, i1, g0, g1 = _run_router(inputs, Wr, br, expert_embeddings)
    row_token, row_gate, blk_e, pos0, pos1 = _route_metadata(i0, i1, g0, g1)
    xs = _sc_gather(inputs, row_token)
    ys = _run_mlp(xs, row_gate, blk_e, W1, b1, W2, b2, W3, b3)
    return _sc_combine(ys, pos0, pos1)


# speedup vs baseline: 1.1712x; 1.1712x over previous
import functools

import jax
import jax.numpy as jnp
from jax import lax
from jax.experimental import pallas as pl
from jax.experimental.pallas import tpu as pltpu
from jax.experimental.pallas import tpu_sc as plsc

N = 2048
D = 1024
E = 8
K = 2
EMB = 1024
H1 = 2048
H2 = 1024
DO = 1024

RBLK = 256
BLK = 128
MAXPAD = 5120
NBLK = MAXPAD // BLK

NC, NS = 2, 16
NW = NC * NS


def _router_body(x_ref, wr_ref, br_ref, emb_ref, i0_ref, i1_ref, g0_ref,
                 g1_ref):
    x = x_ref[...]
    q = jnp.dot(x.astype(jnp.bfloat16), wr_ref[...].astype(jnp.bfloat16),
                preferred_element_type=jnp.float32)
    q = q + br_ref[...]
    cols = []
    for e in range(E):
        de = q - emb_ref[e, :][None, :]
        cols.append(-jnp.sum(de * de, axis=1, keepdims=True))
    scores = jnp.concatenate(cols, axis=1)
    iota = lax.broadcasted_iota(jnp.int32, scores.shape, 1)
    neginf = jnp.float32(-jnp.inf)
    m1 = jnp.max(scores, axis=1, keepdims=True)
    a1 = jnp.min(jnp.where(scores == m1, iota, E), axis=1, keepdims=True)
    masked = jnp.where(iota == a1, neginf, scores)
    m2 = jnp.max(masked, axis=1, keepdims=True)
    a2 = jnp.min(jnp.where(masked == m2, iota, E), axis=1, keepdims=True)
    g = 1.0 / (1.0 + jnp.exp(m2 - m1))
    i0_ref[...] = a1
    i1_ref[...] = a2
    g0_ref[...] = g
    g1_ref[...] = 1.0 - g


def _run_router(x, wr, br, emb):
    out_shapes = (
        jax.ShapeDtypeStruct((N, 1), jnp.int32),
        jax.ShapeDtypeStruct((N, 1), jnp.int32),
        jax.ShapeDtypeStruct((N, 1), jnp.float32),
        jax.ShapeDtypeStruct((N, 1), jnp.float32),
    )
    ospec = pl.BlockSpec((RBLK, 1), lambda i: (i, 0))
    return pl.pallas_call(
        _router_body,
        grid=(N // RBLK,),
        in_specs=[
            pl.BlockSpec((RBLK, D), lambda i: (i, 0)),
            pl.BlockSpec((D, EMB), lambda i: (0, 0)),
            pl.BlockSpec((1, EMB), lambda i: (0, 0)),
            pl.BlockSpec((E, EMB), lambda i: (0, 0)),
        ],
        out_specs=(ospec, ospec, ospec, ospec),
        out_shape=out_shapes,
    )(x, wr, br.reshape(1, EMB), emb)


def _route_metadata(i0, i1, g0, g1):
    flat_e = jnp.concatenate([i0, i1], axis=1).reshape(N * K)
    flat_g = jnp.concatenate([g0, g1], axis=1).reshape(N * K)
    oh = (flat_e[:, None] == jnp.arange(E)[None, :]).astype(jnp.int32)
    cum = jnp.cumsum(oh, axis=0)
    counts = cum[-1]
    rank = jnp.take_along_axis(cum, flat_e[:, None], axis=1).reshape(-1) - 1
    padded = ((counts + BLK - 1) // BLK) * BLK
    cum_pad = jnp.cumsum(padded)
    pad_start = cum_pad - padded
    padded_pos = (pad_start[flat_e] + rank).astype(jnp.int32)
    row_gate = jnp.zeros((MAXPAD,), jnp.float32).at[padded_pos].set(flat_g)
    blk_e = jnp.searchsorted(cum_pad, jnp.arange(NBLK) * BLK, side="right")
    blk_e = jnp.minimum(blk_e, E - 1).astype(jnp.int32)
    pp = padded_pos.reshape(N, K)
    scat_idx = jnp.transpose(pp.reshape(NW, N // NW, K), (0, 2, 1))
    return row_gate, blk_e, scat_idx, pp[:, 0], pp[:, 1]


def _sc_scatter(x, scat_idx):
    tok_per_w = N // NW
    mesh = plsc.VectorSubcoreMesh(core_axis_name="c", subcore_axis_name="s",
                                  num_cores=NC, num_subcores=NS)

    @functools.partial(
        pl.kernel, mesh=mesh,
        out_type=jax.ShapeDtypeStruct((MAXPAD, D), jnp.float32),
        scratch_types=[
            pltpu.VMEM((K, tok_per_w), jnp.int32),
            pltpu.VMEM((tok_per_w, D), jnp.float32),
            pltpu.SemaphoreType.DMA,
            pltpu.SemaphoreType.DMA,
            pltpu.SemaphoreType.DMA,
        ],
    )
    def k(x_hbm, idx_hbm, out_hbm, idx_v, rows_v, sg, s0, s1):
        wid = lax.axis_index("s") * NC + lax.axis_index("c")
        base = wid * tok_per_w
        pltpu.sync_copy(idx_hbm.at[wid], idx_v)
        pltpu.async_copy(x_hbm.at[pl.ds(base, tok_per_w)], rows_v, sg).wait()
        d0 = pltpu.async_copy(rows_v, out_hbm.at[idx_v.at[0]], s0)
        d1 = pltpu.async_copy(rows_v, out_hbm.at[idx_v.at[1]], s1)
        d0.wait()
        d1.wait()

    return k(x, scat_idx)


def _mlp_body(be_ref, xs_ref, gate_ref, w1_ref, b1_ref, w2_ref, b2_ref,
              w3_ref, b3_ref, out_ref):
    bf = jnp.bfloat16
    x = xs_ref[...]
    h = jnp.dot(x.astype(bf), w1_ref[0].astype(bf),
                preferred_element_type=jnp.float32)
    h = jnp.maximum(h + b1_ref[0], 0.0)
    h = jnp.dot(h.astype(bf), w2_ref[0].astype(bf),
                preferred_element_type=jnp.float32)
    h = jnp.maximum(h + b2_ref[0], 0.0)
    y = jnp.dot(h.astype(bf), w3_ref[0].astype(bf),
                preferred_element_type=jnp.float32)
    y = y + b3_ref[0]
    out_ref[...] = y * gate_ref[...]


def _run_mlp(xs, row_gate, blk_e, w1, b1, w2, b2, w3, b3):
    grid_spec = pltpu.PrefetchScalarGridSpec(
        num_scalar_prefetch=1,
        grid=(NBLK,),
        in_specs=[
            pl.BlockSpec((BLK, D), lambda i, be: (i, 0)),
            pl.BlockSpec((BLK, 1), lambda i, be: (i, 0)),
            pl.BlockSpec((1, D, H1), lambda i, be: (be[i], 0, 0)),
            pl.BlockSpec((1, 1, H1), lambda i, be: (be[i], 0, 0)),
            pl.BlockSpec((1, H1, H2), lambda i, be: (be[i], 0, 0)),
            pl.BlockSpec((1, 1, H2), lambda i, be: (be[i], 0, 0)),
            pl.BlockSpec((1, H2, DO), lambda i, be: (be[i], 0, 0)),
            pl.BlockSpec((1, 1, DO), lambda i, be: (be[i], 0, 0)),
        ],
        out_specs=pl.BlockSpec((BLK, DO), lambda i, be: (i, 0)),
    )
    return pl.pallas_call(
        _mlp_body,
        grid_spec=grid_spec,
        out_shape=jax.ShapeDtypeStruct((MAXPAD, DO), jnp.float32),
    )(blk_e, xs, row_gate.reshape(MAXPAD, 1),
      w1, b1.reshape(E, 1, H1), w2, b2.reshape(E, 1, H2),
      w3, b3.reshape(E, 1, DO))


def _sc_combine(ys, pos0, pos1):
    tok_per_w = N // NW
    ch = 16
    mesh = plsc.VectorSubcoreMesh(core_axis_name="c", subcore_axis_name="s",
                                  num_cores=NC, num_subcores=NS)

    @functools.partial(
        pl.kernel, mesh=mesh,
        out_type=jax.ShapeDtypeStruct((N, DO), jnp.float32),
        scratch_types=[
            pltpu.VMEM((tok_per_w,), jnp.int32),
            pltpu.VMEM((tok_per_w,), jnp.int32),
            pltpu.VMEM((ch, DO), jnp.float32),
            pltpu.VMEM((ch, DO), jnp.float32),
            pltpu.VMEM((ch, DO), jnp.float32),
            pltpu.VMEM((ch, DO), jnp.float32),
            pltpu.SemaphoreType.DMA,
            pltpu.SemaphoreType.DMA,
            pltpu.SemaphoreType.DMA,
            pltpu.SemaphoreType.DMA,
            pltpu.SemaphoreType.DMA,
            pltpu.SemaphoreType.DMA,
        ],
    )
    def k(ys_hbm, p0_hbm, p1_hbm, out_hbm, i0_v, i1_v, a0, a1, b0, b1,
          ga0, ga1, gb0, gb1, s0, s1):
        wid = lax.axis_index("s") * NC + lax.axis_index("c")
        base = wid * tok_per_w
        pltpu.sync_copy(p0_hbm.at[pl.ds(base, tok_per_w)], i0_v)
        pltpu.sync_copy(p1_hbm.at[pl.ds(base, tok_per_w)], i1_v)
        abufs, bbufs = (a0, a1), (b0, b1)
        gasems, gbsems, ssems = (ga0, ga1), (gb0, gb1), (s0, s1)
        nch = tok_per_w // ch
        d0 = pltpu.async_copy(ys_hbm.at[i0_v.at[pl.ds(0, ch)]], a0, ga0)
        d1 = pltpu.async_copy(ys_hbm.at[i1_v.at[pl.ds(0, ch)]], b0, gb0)
        stores = [None, None]
        for c in range(nch):
            b = c & 1
            nd0 = nd1 = None
            if c + 1 < nch:
                ob = (c + 1) & 1
                if stores[ob] is not None:
                    stores[ob].wait()
                sl = pl.ds((c + 1) * ch, ch)
                nd0 = pltpu.async_copy(ys_hbm.at[i0_v.at[sl]], abufs[ob],
                                       gasems[ob])
                nd1 = pltpu.async_copy(ys_hbm.at[i1_v.at[sl]], bbufs[ob],
                                       gbsems[ob])
            d0.wait()
            d1.wait()
            a_ref, b_ref = abufs[b], bbufs[b]

            def row_body(r, _, a_ref=a_ref, b_ref=b_ref):
                for cc in range(DO // 16):
                    s = pl.ds(cc * 16, 16)
                    a_ref[r, s] = a_ref[r, s] + b_ref[r, s]
                return 0

            lax.fori_loop(0, ch, row_body, 0)
            stores[b] = pltpu.async_copy(a_ref,
                                         out_hbm.at[pl.ds(base + c * ch, ch)],
                                         ssems[b])
            d0, d1 = nd0, nd1
        stores[0].wait()
        stores[1].wait()

    return k(ys, pos0, pos1)


def kernel(inputs, Wr, br, expert_embeddings, W1, b1, W2, b2, W3, b3):
    i0, i1, g0, g1 = _run_router(inputs, Wr, br, expert_embeddings)
    row_gate, blk_e, scat_idx, pos0, pos1 = _route_metadata(i0, i1, g0, g1)
    xs = _sc_scatter(inputs, scat_idx)
    ys = _run_mlp(xs, row_gate, blk_e, W1, b1, W2, b2, W3, b3)
    return _sc_combine(ys, pos0, pos1)


# speedup vs baseline: 1.3515x; 1.1540x over previous
import functools

import jax
import jax.numpy as jnp
from jax import lax
from jax.experimental import pallas as pl
from jax.experimental.pallas import tpu as pltpu
from jax.experimental.pallas import tpu_sc as plsc

N = 2048
D = 1024
E = 8
K = 2
EMB = 1024
H1 = 2048
H2 = 1024
DO = 1024

RBLK = 256
BLK = 128
MAXPAD = 5120
NBLK = MAXPAD // BLK

NC, NS = 2, 16
NW = NC * NS


def _router_body(x_ref, wr_ref, br_ref, emb_ref, i0_ref, i1_ref, g0_ref,
                 g1_ref):
    x = x_ref[...]
    q = jnp.dot(x.astype(jnp.bfloat16), wr_ref[...].astype(jnp.bfloat16),
                preferred_element_type=jnp.float32)
    q = q + br_ref[...]
    cols = []
    for e in range(E):
        de = q - emb_ref[e, :][None, :]
        cols.append(-jnp.sum(de * de, axis=1, keepdims=True))
    scores = jnp.concatenate(cols, axis=1)
    iota = lax.broadcasted_iota(jnp.int32, scores.shape, 1)
    neginf = jnp.float32(-jnp.inf)
    m1 = jnp.max(scores, axis=1, keepdims=True)
    a1 = jnp.min(jnp.where(scores == m1, iota, E), axis=1, keepdims=True)
    masked = jnp.where(iota == a1, neginf, scores)
    m2 = jnp.max(masked, axis=1, keepdims=True)
    a2 = jnp.min(jnp.where(masked == m2, iota, E), axis=1, keepdims=True)
    g = 1.0 / (1.0 + jnp.exp(m2 - m1))
    i0_ref[...] = a1
    i1_ref[...] = a2
    g0_ref[...] = g
    g1_ref[...] = 1.0 - g


def _run_router(x, wr, br, emb):
    out_shapes = (
        jax.ShapeDtypeStruct((N, 1), jnp.int32),
        jax.ShapeDtypeStruct((N, 1), jnp.int32),
        jax.ShapeDtypeStruct((N, 1), jnp.float32),
        jax.ShapeDtypeStruct((N, 1), jnp.float32),
    )
    ospec = pl.BlockSpec((RBLK, 1), lambda i: (i, 0))
    return pl.pallas_call(
        _router_body,
        grid=(N // RBLK,),
        in_specs=[
            pl.BlockSpec((RBLK, D), lambda i: (i, 0)),
            pl.BlockSpec((D, EMB), lambda i: (0, 0)),
            pl.BlockSpec((1, EMB), lambda i: (0, 0)),
            pl.BlockSpec((E, EMB), lambda i: (0, 0)),
        ],
        out_specs=(ospec, ospec, ospec, ospec),
        out_shape=out_shapes,
    )(x, wr, br.reshape(1, EMB), emb)


def _route_metadata(i0, i1, g0, g1):
    flat_e = jnp.concatenate([i0, i1], axis=1).reshape(N * K)
    flat_g = jnp.concatenate([g0, g1], axis=1).reshape(N * K)
    oh = (flat_e[:, None] == jnp.arange(E)[None, :]).astype(jnp.int32)
    cum = jnp.cumsum(oh, axis=0)
    counts = cum[-1]
    rank = jnp.take_along_axis(cum, flat_e[:, None], axis=1).reshape(-1) - 1
    padded = ((counts + BLK - 1) // BLK) * BLK
    cum_pad = jnp.cumsum(padded)
    pad_start = cum_pad - padded
    padded_pos = (pad_start[flat_e] + rank).astype(jnp.int32)
    starts = jnp.arange(NBLK, dtype=jnp.int32) * BLK
    blk_e = jnp.sum((starts[:, None] >= cum_pad[None, :]).astype(jnp.int32),
                    axis=1)
    blk_e = jnp.minimum(blk_e, E - 1).astype(jnp.int32)
    pp = padded_pos.reshape(N, K)
    scat_idx = jnp.transpose(pp.reshape(NW, N // NW, K), (0, 2, 1))
    gates = flat_g.reshape(N, K)
    g0rep = jnp.broadcast_to(gates[:, 0:1], (N, 16))
    g1rep = jnp.broadcast_to(gates[:, 1:2], (N, 16))
    return g0rep, g1rep, blk_e, scat_idx, pp[:, 0], pp[:, 1]


def _sc_scatter(x, scat_idx):
    tok_per_w = N // NW
    mesh = plsc.VectorSubcoreMesh(core_axis_name="c", subcore_axis_name="s",
                                  num_cores=NC, num_subcores=NS)

    @functools.partial(
        pl.kernel, mesh=mesh,
        out_type=jax.ShapeDtypeStruct((MAXPAD, D), jnp.float32),
        scratch_types=[
            pltpu.VMEM((K, tok_per_w), jnp.int32),
            pltpu.VMEM((tok_per_w, D), jnp.float32),
            pltpu.SemaphoreType.DMA,
            pltpu.SemaphoreType.DMA,
            pltpu.SemaphoreType.DMA,
        ],
    )
    def k(x_hbm, idx_hbm, out_hbm, idx_v, rows_v, sg, s0, s1):
        wid = lax.axis_index("s") * NC + lax.axis_index("c")
        base = wid * tok_per_w
        pltpu.sync_copy(idx_hbm.at[wid], idx_v)
        pltpu.async_copy(x_hbm.at[pl.ds(base, tok_per_w)], rows_v, sg).wait()
        d0 = pltpu.async_copy(rows_v, out_hbm.at[idx_v.at[0]], s0)
        d1 = pltpu.async_copy(rows_v, out_hbm.at[idx_v.at[1]], s1)
        d0.wait()
        d1.wait()

    return k(x, scat_idx)


def _mlp_body(be_ref, xs_ref, w1_ref, b1_ref, w2_ref, b2_ref,
              w3_ref, b3_ref, out_ref):
    bf = jnp.bfloat16
    x = xs_ref[...]
    h = jnp.dot(x.astype(bf), w1_ref[0].astype(bf),
                preferred_element_type=jnp.float32)
    h = jnp.maximum(h + b1_ref[0], 0.0)
    h = jnp.dot(h.astype(bf), w2_ref[0].astype(bf),
                preferred_element_type=jnp.float32)
    h = jnp.maximum(h + b2_ref[0], 0.0)
    y = jnp.dot(h.astype(bf), w3_ref[0].astype(bf),
                preferred_element_type=jnp.float32)
    out_ref[...] = y + b3_ref[0]


def _run_mlp(xs, blk_e, w1, b1, w2, b2, w3, b3):
    grid_spec = pltpu.PrefetchScalarGridSpec(
        num_scalar_prefetch=1,
        grid=(NBLK,),
        in_specs=[
            pl.BlockSpec((BLK, D), lambda i, be: (i, 0)),
            pl.BlockSpec((1, D, H1), lambda i, be: (be[i], 0, 0)),
            pl.BlockSpec((1, 1, H1), lambda i, be: (be[i], 0, 0)),
            pl.BlockSpec((1, H1, H2), lambda i, be: (be[i], 0, 0)),
            pl.BlockSpec((1, 1, H2), lambda i, be: (be[i], 0, 0)),
            pl.BlockSpec((1, H2, DO), lambda i, be: (be[i], 0, 0)),
            pl.BlockSpec((1, 1, DO), lambda i, be: (be[i], 0, 0)),
        ],
        out_specs=pl.BlockSpec((BLK, DO), lambda i, be: (i, 0)),
    )
    return pl.pallas_call(
        _mlp_body,
        grid_spec=grid_spec,
        out_shape=jax.ShapeDtypeStruct((MAXPAD, DO), jnp.float32),
    )(blk_e, xs,
      w1, b1.reshape(E, 1, H1), w2, b2.reshape(E, 1, H2),
      w3, b3.reshape(E, 1, DO))


def _sc_combine(ys, pos0, pos1, g0rep, g1rep):
    tok_per_w = N // NW
    ch = 16
    mesh = plsc.VectorSubcoreMesh(core_axis_name="c", subcore_axis_name="s",
                                  num_cores=NC, num_subcores=NS)

    @functools.partial(
        pl.kernel, mesh=mesh,
        out_type=jax.ShapeDtypeStruct((N, DO), jnp.float32),
        scratch_types=[
            pltpu.VMEM((tok_per_w,), jnp.int32),
            pltpu.VMEM((tok_per_w,), jnp.int32),
            pltpu.VMEM((tok_per_w, 16), jnp.float32),
            pltpu.VMEM((tok_per_w, 16), jnp.float32),
            pltpu.VMEM((ch, DO), jnp.float32),
            pltpu.VMEM((ch, DO), jnp.float32),
            pltpu.VMEM((ch, DO), jnp.float32),
            pltpu.VMEM((ch, DO), jnp.float32),
            pltpu.SemaphoreType.DMA,
            pltpu.SemaphoreType.DMA,
            pltpu.SemaphoreType.DMA,
            pltpu.SemaphoreType.DMA,
            pltpu.SemaphoreType.DMA,
            pltpu.SemaphoreType.DMA,
        ],
    )
    def k(ys_hbm, p0_hbm, p1_hbm, g0_hbm, g1_hbm, out_hbm, i0_v, i1_v,
          g0_v, g1_v, a0, a1, b0, b1, ga0, ga1, gb0, gb1, s0, s1):
        wid = lax.axis_index("s") * NC + lax.axis_index("c")
        base = wid * tok_per_w
        pltpu.sync_copy(p0_hbm.at[pl.ds(base, tok_per_w)], i0_v)
        pltpu.sync_copy(p1_hbm.at[pl.ds(base, tok_per_w)], i1_v)
        pltpu.sync_copy(g0_hbm.at[pl.ds(base, tok_per_w)], g0_v)
        pltpu.sync_copy(g1_hbm.at[pl.ds(base, tok_per_w)], g1_v)
        abufs, bbufs = (a0, a1), (b0, b1)
        gasems, gbsems, ssems = (ga0, ga1), (gb0, gb1), (s0, s1)
        nch = tok_per_w // ch
        d0 = pltpu.async_copy(ys_hbm.at[i0_v.at[pl.ds(0, ch)]], a0, ga0)
        d1 = pltpu.async_copy(ys_hbm.at[i1_v.at[pl.ds(0, ch)]], b0, gb0)
        stores = [None, None]
        for c in range(nch):
            b = c & 1
            nd0 = nd1 = None
            if c + 1 < nch:
                ob = (c + 1) & 1
                if stores[ob] is not None:
                    stores[ob].wait()
                sl = pl.ds((c + 1) * ch, ch)
                nd0 = pltpu.async_copy(ys_hbm.at[i0_v.at[sl]], abufs[ob],
                                       gasems[ob])
                nd1 = pltpu.async_copy(ys_hbm.at[i1_v.at[sl]], bbufs[ob],
                                       gbsems[ob])
            d0.wait()
            d1.wait()
            a_ref, b_ref = abufs[b], bbufs[b]

            def row_body(r, _, a_ref=a_ref, b_ref=b_ref, c=c):
                ga = g0_v[c * ch + r, :]
                gb = g1_v[c * ch + r, :]
                for cc in range(DO // 16):
                    s = pl.ds(cc * 16, 16)
                    a_ref[r, s] = a_ref[r, s] * ga + b_ref[r, s] * gb
                return 0

            lax.fori_loop(0, ch, row_body, 0)
            stores[b] = pltpu.async_copy(a_ref,
                                         out_hbm.at[pl.ds(base + c * ch, ch)],
                                         ssems[b])
            d0, d1 = nd0, nd1
        stores[0].wait()
        stores[1].wait()

    return k(ys, pos0, pos1, g0rep, g1rep)


def kernel(inputs, Wr, br, expert_embeddings, W1, b1, W2, b2, W3, b3):
    i0, i1, g0, g1 = _run_router(inputs, Wr, br, expert_embeddings)
    g0rep, g1rep, blk_e, scat_idx, pos0, pos1 = _route_metadata(i0, i1, g0, g1)
    xs = _sc_scatter(inputs, scat_idx)
    ys = _run_mlp(xs, blk_e, W1, b1, W2, b2, W3, b3)
    return _sc_combine(ys, pos0, pos1, g0rep, g1rep)


# speedup vs baseline: 1.3672x; 1.0116x over previous
import functools

import jax
import jax.numpy as jnp
from jax import lax
from jax.experimental import pallas as pl
from jax.experimental.pallas import tpu as pltpu
from jax.experimental.pallas import tpu_sc as plsc

N = 2048
D = 1024
E = 8
K = 2
EMB = 1024
H1 = 2048
H2 = 1024
DO = 1024

RBLK = 256
BLK = 128
MAXPAD = 5120
NBLK = MAXPAD // BLK

NC, NS = 2, 16
NW = NC * NS


def _router_body(x_ref, wr_ref, br_ref, emb_ref, i0_ref, i1_ref, g0_ref,
                 g1_ref, r0_ref, r1_ref, cnt_ref, run_ref):
    @pl.when(pl.program_id(0) == 0)
    def _():
        run_ref[...] = jnp.zeros_like(run_ref)

    x = x_ref[...]
    q = jnp.dot(x.astype(jnp.bfloat16), wr_ref[...].astype(jnp.bfloat16),
                preferred_element_type=jnp.float32)
    q = q + br_ref[...]
    cols = []
    for e in range(E):
        de = q - emb_ref[e, :][None, :]
        cols.append(-jnp.sum(de * de, axis=1, keepdims=True))
    scores = jnp.concatenate(cols, axis=1)
    iota = lax.broadcasted_iota(jnp.int32, scores.shape, 1)
    neginf = jnp.float32(-jnp.inf)
    m1 = jnp.max(scores, axis=1, keepdims=True)
    a1 = jnp.min(jnp.where(scores == m1, iota, E), axis=1, keepdims=True)
    masked = jnp.where(iota == a1, neginf, scores)
    m2 = jnp.max(masked, axis=1, keepdims=True)
    a2 = jnp.min(jnp.where(masked == m2, iota, E), axis=1, keepdims=True)
    g = 1.0 / (1.0 + jnp.exp(m2 - m1))
    i0_ref[...] = a1
    i1_ref[...] = a2
    g0_ref[...] = g
    g1_ref[...] = 1.0 - g
    oh = (jnp.where(iota == a1, 1.0, 0.0)
          + jnp.where(iota == a2, 1.0, 0.0))
    rr = lax.broadcasted_iota(jnp.int32, (RBLK, RBLK), 0)
    cc = lax.broadcasted_iota(jnp.int32, (RBLK, RBLK), 1)
    ltri = jnp.where(cc < rr, 1.0, 0.0).astype(jnp.bfloat16)
    prefix = jnp.dot(ltri, oh.astype(jnp.bfloat16),
                     preferred_element_type=jnp.float32)
    rankq = prefix + run_ref[0:1, 0:E]
    r0_ref[...] = jnp.sum(jnp.where(iota == a1, rankq, 0.0), axis=1,
                          keepdims=True).astype(jnp.int32)
    r1_ref[...] = jnp.sum(jnp.where(iota == a2, rankq, 0.0), axis=1,
                          keepdims=True).astype(jnp.int32)
    run_ref[0:1, 0:E] = run_ref[0:1, 0:E] + jnp.sum(oh, axis=0, keepdims=True)
    cnt_ref[...] = run_ref[0:1, 0:E]


def _run_router(x, wr, br, emb):
    out_shapes = (
        jax.ShapeDtypeStruct((N, 1), jnp.int32),
        jax.ShapeDtypeStruct((N, 1), jnp.int32),
        jax.ShapeDtypeStruct((N, 1), jnp.float32),
        jax.ShapeDtypeStruct((N, 1), jnp.float32),
        jax.ShapeDtypeStruct((N, 1), jnp.int32),
        jax.ShapeDtypeStruct((N, 1), jnp.int32),
        jax.ShapeDtypeStruct((1, E), jnp.float32),
    )
    ospec = pl.BlockSpec((RBLK, 1), lambda i: (i, 0))
    cspec = pl.BlockSpec((1, E), lambda i: (0, 0))
    return pl.pallas_call(
        _router_body,
        grid=(N // RBLK,),
        in_specs=[
            pl.BlockSpec((RBLK, D), lambda i: (i, 0)),
            pl.BlockSpec((D, EMB), lambda i: (0, 0)),
            pl.BlockSpec((1, EMB), lambda i: (0, 0)),
            pl.BlockSpec((E, EMB), lambda i: (0, 0)),
        ],
        out_specs=(ospec, ospec, ospec, ospec, ospec, ospec, cspec),
        out_shape=out_shapes,
        scratch_shapes=[pltpu.VMEM((8, 128), jnp.float32)],
    )(x, wr, br.reshape(1, EMB), emb)


def _route_metadata(i0, i1, g0, g1, r0, r1, cnt):
    counts = cnt.reshape(E).astype(jnp.int32)
    padded = ((counts + BLK - 1) // BLK) * BLK
    cum_pad = jnp.cumsum(padded)
    pad_start = cum_pad - padded
    pos0 = (pad_start[i0.reshape(N)] + r0.reshape(N)).astype(jnp.int32)
    pos1 = (pad_start[i1.reshape(N)] + r1.reshape(N)).astype(jnp.int32)
    starts = jnp.arange(NBLK, dtype=jnp.int32) * BLK
    blk_e = jnp.sum((starts[:, None] >= cum_pad[None, :]).astype(jnp.int32),
                    axis=1)
    blk_e = jnp.minimum(blk_e, E - 1).astype(jnp.int32)
    pp = jnp.stack([pos0, pos1], axis=1)
    scat_idx = jnp.transpose(pp.reshape(NW, N // NW, K), (0, 2, 1))
    g0rep = jnp.broadcast_to(g0, (N, 16))
    g1rep = jnp.broadcast_to(g1, (N, 16))
    return g0rep, g1rep, blk_e, scat_idx, pos0, pos1


def _sc_scatter(x, scat_idx):
    tok_per_w = N // NW
    mesh = plsc.VectorSubcoreMesh(core_axis_name="c", subcore_axis_name="s",
                                  num_cores=NC, num_subcores=NS)

    @functools.partial(
        pl.kernel, mesh=mesh,
        out_type=jax.ShapeDtypeStruct((MAXPAD, D), jnp.float32),
        scratch_types=[
            pltpu.VMEM((K, tok_per_w), jnp.int32),
            pltpu.VMEM((tok_per_w, D), jnp.float32),
            pltpu.SemaphoreType.DMA,
            pltpu.SemaphoreType.DMA,
            pltpu.SemaphoreType.DMA,
        ],
    )
    def k(x_hbm, idx_hbm, out_hbm, idx_v, rows_v, sg, s0, s1):
        wid = lax.axis_index("s") * NC + lax.axis_index("c")
        base = wid * tok_per_w
        pltpu.sync_copy(idx_hbm.at[wid], idx_v)
        pltpu.async_copy(x_hbm.at[pl.ds(base, tok_per_w)], rows_v, sg).wait()
        d0 = pltpu.async_copy(rows_v, out_hbm.at[idx_v.at[0]], s0)
        d1 = pltpu.async_copy(rows_v, out_hbm.at[idx_v.at[1]], s1)
        d0.wait()
        d1.wait()

    return k(x, scat_idx)


def _mlp_body(be_ref, xs_ref, w1_ref, b1_ref, w2_ref, b2_ref,
              w3_ref, b3_ref, out_ref):
    bf = jnp.bfloat16
    x = xs_ref[...]
    h = jnp.dot(x.astype(bf), w1_ref[0].astype(bf),
                preferred_element_type=jnp.float32)
    h = jnp.maximum(h + b1_ref[0], 0.0)
    h = jnp.dot(h.astype(bf), w2_ref[0].astype(bf),
                preferred_element_type=jnp.float32)
    h = jnp.maximum(h + b2_ref[0], 0.0)
    y = jnp.dot(h.astype(bf), w3_ref[0].astype(bf),
                preferred_element_type=jnp.float32)
    out_ref[...] = y + b3_ref[0]


def _run_mlp(xs, blk_e, w1, b1, w2, b2, w3, b3):
    grid_spec = pltpu.PrefetchScalarGridSpec(
        num_scalar_prefetch=1,
        grid=(NBLK,),
        in_specs=[
            pl.BlockSpec((BLK, D), lambda i, be: (i, 0)),
            pl.BlockSpec((1, D, H1), lambda i, be: (be[i], 0, 0)),
            pl.BlockSpec((1, 1, H1), lambda i, be: (be[i], 0, 0)),
            pl.BlockSpec((1, H1, H2), lambda i, be: (be[i], 0, 0)),
            pl.BlockSpec((1, 1, H2), lambda i, be: (be[i], 0, 0)),
            pl.BlockSpec((1, H2, DO), lambda i, be: (be[i], 0, 0)),
            pl.BlockSpec((1, 1, DO), lambda i, be: (be[i], 0, 0)),
        ],
        out_specs=pl.BlockSpec((BLK, DO), lambda i, be: (i, 0)),
    )
    return pl.pallas_call(
        _mlp_body,
        grid_spec=grid_spec,
        out_shape=jax.ShapeDtypeStruct((MAXPAD, DO), jnp.float32),
    )(blk_e, xs,
      w1, b1.reshape(E, 1, H1), w2, b2.reshape(E, 1, H2),
      w3, b3.reshape(E, 1, DO))


def _sc_combine(ys, pos0, pos1, g0rep, g1rep):
    tok_per_w = N // NW
    ch = 16
    mesh = plsc.VectorSubcoreMesh(core_axis_name="c", subcore_axis_name="s",
                                  num_cores=NC, num_subcores=NS)

    @functools.partial(
        pl.kernel, mesh=mesh,
        out_type=jax.ShapeDtypeStruct((N, DO), jnp.float32),
        scratch_types=[
            pltpu.VMEM((tok_per_w,), jnp.int32),
            pltpu.VMEM((tok_per_w,), jnp.int32),
            pltpu.VMEM((tok_per_w, 16), jnp.float32),
            pltpu.VMEM((tok_per_w, 16), jnp.float32),
            pltpu.VMEM((ch, DO), jnp.float32),
            pltpu.VMEM((ch, DO), jnp.float32),
            pltpu.VMEM((ch, DO), jnp.float32),
            pltpu.VMEM((ch, DO), jnp.float32),
            pltpu.SemaphoreType.DMA,
            pltpu.SemaphoreType.DMA,
            pltpu.SemaphoreType.DMA,
            pltpu.SemaphoreType.DMA,
            pltpu.SemaphoreType.DMA,
            pltpu.SemaphoreType.DMA,
        ],
    )
    def k(ys_hbm, p0_hbm, p1_hbm, g0_hbm, g1_hbm, out_hbm, i0_v, i1_v,
          g0_v, g1_v, a0, a1, b0, b1, ga0, ga1, gb0, gb1, s0, s1):
        wid = lax.axis_index("s") * NC + lax.axis_index("c")
        base = wid * tok_per_w
        pltpu.sync_copy(p0_hbm.at[pl.ds(base, tok_per_w)], i0_v)
        pltpu.sync_copy(p1_hbm.at[pl.ds(base, tok_per_w)], i1_v)
        pltpu.sync_copy(g0_hbm.at[pl.ds(base, tok_per_w)], g0_v)
        pltpu.sync_copy(g1_hbm.at[pl.ds(base, tok_per_w)], g1_v)
        abufs, bbufs = (a0, a1), (b0, b1)
        gasems, gbsems, ssems = (ga0, ga1), (gb0, gb1), (s0, s1)
        nch = tok_per_w // ch
        d0 = pltpu.async_copy(ys_hbm.at[i0_v.at[pl.ds(0, ch)]], a0, ga0)
        d1 = pltpu.async_copy(ys_hbm.at[i1_v.at[pl.ds(0, ch)]], b0, gb0)
        stores = [None, None]
        for c in range(nch):
            b = c & 1
            nd0 = nd1 = None
            if c + 1 < nch:
                ob = (c + 1) & 1
                if stores[ob] is not None:
                    stores[ob].wait()
                sl = pl.ds((c + 1) * ch, ch)
                nd0 = pltpu.async_copy(ys_hbm.at[i0_v.at[sl]], abufs[ob],
                                       gasems[ob])
                nd1 = pltpu.async_copy(ys_hbm.at[i1_v.at[sl]], bbufs[ob],
                                       gbsems[ob])
            d0.wait()
            d1.wait()
            a_ref, b_ref = abufs[b], bbufs[b]

            def row_body(r, _, a_ref=a_ref, b_ref=b_ref, c=c):
                ga = g0_v[c * ch + r, :]
                gb = g1_v[c * ch + r, :]
                for cc in range(DO // 16):
                    s = pl.ds(cc * 16, 16)
                    a_ref[r, s] = a_ref[r, s] * ga + b_ref[r, s] * gb
                return 0

            lax.fori_loop(0, ch, row_body, 0)
            stores[b] = pltpu.async_copy(a_ref,
                                         out_hbm.at[pl.ds(base + c * ch, ch)],
                                         ssems[b])
            d0, d1 = nd0, nd1
        stores[0].wait()
        stores[1].wait()

    return k(ys, pos0, pos1, g0rep, g1rep)


def kernel(inputs, Wr, br, expert_embeddings, W1, b1, W2, b2, W3, b3):
    i0, i1, g0, g1, r0, r1, cnt = _run_router(inputs, Wr, br,
                                              expert_embeddings)
    g0rep, g1rep, blk_e, scat_idx, pos0, pos1 = _route_metadata(
        i0, i1, g0, g1, r0, r1, cnt)
    xs = _sc_scatter(inputs, scat_idx)
    ys = _run_mlp(xs, blk_e, W1, b1, W2, b2, W3, b3)
    return _sc_combine(ys, pos0, pos1, g0rep, g1rep)


# speedup vs baseline: 1.4108x; 1.0319x over previous
import functools

import jax
import jax.numpy as jnp
from jax import lax
from jax.experimental import pallas as pl
from jax.experimental.pallas import tpu as pltpu
from jax.experimental.pallas import tpu_sc as plsc

N = 2048
D = 1024
E = 8
K = 2
EMB = 1024
H1 = 2048
H2 = 1024
DO = 1024

RBLK = 256
BLK = 128
MAXPAD = 5120
NBLK = MAXPAD // BLK

NC, NS = 2, 16
NW = NC * NS


def _router_body(x_ref, wr_ref, br_ref, emb_ref, i0_ref, i1_ref, g0_ref,
                 g1_ref, r0_ref, r1_ref, cnt_ref, run_ref):
    @pl.when(pl.program_id(0) == 0)
    def _():
        run_ref[...] = jnp.zeros_like(run_ref)

    x = x_ref[...]
    q = jnp.dot(x.astype(jnp.bfloat16), wr_ref[...].astype(jnp.bfloat16),
                preferred_element_type=jnp.float32)
    q = q + br_ref[...]
    cols = []
    for e in range(E):
        de = q - emb_ref[e, :][None, :]
        cols.append(-jnp.sum(de * de, axis=1, keepdims=True))
    scores = jnp.concatenate(cols, axis=1)
    iota = lax.broadcasted_iota(jnp.int32, scores.shape, 1)
    neginf = jnp.float32(-jnp.inf)
    m1 = jnp.max(scores, axis=1, keepdims=True)
    a1 = jnp.min(jnp.where(scores == m1, iota, E), axis=1, keepdims=True)
    masked = jnp.where(iota == a1, neginf, scores)
    m2 = jnp.max(masked, axis=1, keepdims=True)
    a2 = jnp.min(jnp.where(masked == m2, iota, E), axis=1, keepdims=True)
    g = 1.0 / (1.0 + jnp.exp(m2 - m1))
    i0_ref[...] = a1
    i1_ref[...] = a2
    g0_ref[...] = g
    g1_ref[...] = 1.0 - g
    oh = (jnp.where(iota == a1, 1.0, 0.0)
          + jnp.where(iota == a2, 1.0, 0.0))
    rr = lax.broadcasted_iota(jnp.int32, (RBLK, RBLK), 0)
    cc = lax.broadcasted_iota(jnp.int32, (RBLK, RBLK), 1)
    ltri = jnp.where(cc < rr, 1.0, 0.0).astype(jnp.bfloat16)
    prefix = jnp.dot(ltri, oh.astype(jnp.bfloat16),
                     preferred_element_type=jnp.float32)
    rankq = prefix + run_ref[0:1, 0:E]
    r0_ref[...] = jnp.sum(jnp.where(iota == a1, rankq, 0.0), axis=1,
                          keepdims=True).astype(jnp.int32)
    r1_ref[...] = jnp.sum(jnp.where(iota == a2, rankq, 0.0), axis=1,
                          keepdims=True).astype(jnp.int32)
    run_ref[0:1, 0:E] = run_ref[0:1, 0:E] + jnp.sum(oh, axis=0, keepdims=True)
    cnt_ref[...] = run_ref[0:1, 0:E]


def _run_router(x, wr, br, emb):
    out_shapes = (
        jax.ShapeDtypeStruct((N, 1), jnp.int32),
        jax.ShapeDtypeStruct((N, 1), jnp.int32),
        jax.ShapeDtypeStruct((N, 1), jnp.float32),
        jax.ShapeDtypeStruct((N, 1), jnp.float32),
        jax.ShapeDtypeStruct((N, 1), jnp.int32),
        jax.ShapeDtypeStruct((N, 1), jnp.int32),
        jax.ShapeDtypeStruct((1, E), jnp.float32),
    )
    ospec = pl.BlockSpec((RBLK, 1), lambda i: (i, 0))
    cspec = pl.BlockSpec((1, E), lambda i: (0, 0))
    return pl.pallas_call(
        _router_body,
        grid=(N // RBLK,),
        in_specs=[
            pl.BlockSpec((RBLK, D), lambda i: (i, 0)),
            pl.BlockSpec((D, EMB), lambda i: (0, 0)),
            pl.BlockSpec((1, EMB), lambda i: (0, 0)),
            pl.BlockSpec((E, EMB), lambda i: (0, 0)),
        ],
        out_specs=(ospec, ospec, ospec, ospec, ospec, ospec, cspec),
        out_shape=out_shapes,
        scratch_shapes=[pltpu.VMEM((8, 128), jnp.float32)],
    )(x, wr, br.reshape(1, EMB), emb)


def _route_metadata(g0, g1, cnt):
    counts = cnt.reshape(E).astype(jnp.int32)
    padded = ((counts + BLK - 1) // BLK) * BLK
    cum_pad = jnp.cumsum(padded)
    pad_start_b = jnp.broadcast_to((cum_pad - padded)[:, None],
                                   (E, 16)).astype(jnp.int32)
    starts = jnp.arange(NBLK, dtype=jnp.int32) * BLK
    blk_e = jnp.sum((starts[:, None] >= cum_pad[None, :]).astype(jnp.int32),
                    axis=1)
    blk_e = jnp.minimum(blk_e, E - 1).astype(jnp.int32)
    g0rep = jnp.broadcast_to(g0, (N, 16))
    g1rep = jnp.broadcast_to(g1, (N, 16))
    return g0rep, g1rep, blk_e, pad_start_b


def _sc_scatter(x, i0, i1, r0, r1, pad_start_b):
    tok_per_w = N // NW
    mesh = plsc.VectorSubcoreMesh(core_axis_name="c", subcore_axis_name="s",
                                  num_cores=NC, num_subcores=NS)

    @functools.partial(
        pl.kernel, mesh=mesh,
        out_type=(
            jax.ShapeDtypeStruct((MAXPAD, D), jnp.float32),
            jax.ShapeDtypeStruct((N,), jnp.int32),
            jax.ShapeDtypeStruct((N,), jnp.int32),
        ),
        scratch_types=[
            pltpu.VMEM((E, 16), jnp.int32),
            pltpu.VMEM((tok_per_w,), jnp.int32),
            pltpu.VMEM((tok_per_w,), jnp.int32),
            pltpu.VMEM((tok_per_w,), jnp.int32),
            pltpu.VMEM((tok_per_w,), jnp.int32),
            pltpu.VMEM((K, tok_per_w), jnp.int32),
            pltpu.VMEM((tok_per_w, D), jnp.float32),
            pltpu.SemaphoreType.DMA,
            pltpu.SemaphoreType.DMA,
            pltpu.SemaphoreType.DMA,
        ],
    )
    def k(x_hbm, i0_hbm, i1_hbm, r0_hbm, r1_hbm, cnt_hbm, out_hbm, p0_hbm,
          p1_hbm, ps_v, e0_v, e1_v, r0_v, r1_v, idx_v, rows_v,
          sg, s0, s1):
        wid = lax.axis_index("s") * NC + lax.axis_index("c")
        base = wid * tok_per_w
        pltpu.sync_copy(cnt_hbm, ps_v)
        dl = pltpu.async_copy(x_hbm.at[pl.ds(base, tok_per_w)], rows_v, sg)
        pltpu.sync_copy(i0_hbm.at[pl.ds(base, tok_per_w)], e0_v)
        pltpu.sync_copy(i1_hbm.at[pl.ds(base, tok_per_w)], e1_v)
        pltpu.sync_copy(r0_hbm.at[pl.ds(base, tok_per_w)], r0_v)
        pltpu.sync_copy(r1_hbm.at[pl.ds(base, tok_per_w)], r1_v)
        for j in range(tok_per_w // 16):
            sl = pl.ds(j * 16, 16)
            e0 = e0_v[sl]
            e1 = e1_v[sl]
            acc0 = r0_v[sl]
            acc1 = r1_v[sl]
            zero = jnp.zeros((16,), jnp.int32)
            for e in range(E):
                pse = ps_v[e, :]
                acc0 = acc0 + jnp.where(e0 == e, pse, zero)
                acc1 = acc1 + jnp.where(e1 == e, pse, zero)
            idx_v[0, sl] = acc0
            idx_v[1, sl] = acc1
        pltpu.sync_copy(idx_v.at[0], p0_hbm.at[pl.ds(base, tok_per_w)])
        pltpu.sync_copy(idx_v.at[1], p1_hbm.at[pl.ds(base, tok_per_w)])
        dl.wait()
        d0 = pltpu.async_copy(rows_v, out_hbm.at[idx_v.at[0]], s0)
        d1 = pltpu.async_copy(rows_v, out_hbm.at[idx_v.at[1]], s1)
        d0.wait()
        d1.wait()

    return k(x, i0, i1, r0, r1, pad_start_b)


def _mlp_body(be_ref, xs_ref, w1_ref, b1_ref, w2_ref, b2_ref,
              w3_ref, b3_ref, out_ref):
    bf = jnp.bfloat16
    x = xs_ref[...]
    h = jnp.dot(x.astype(bf), w1_ref[0].astype(bf),
                preferred_element_type=jnp.float32)
    h = jnp.maximum(h + b1_ref[0], 0.0)
    h = jnp.dot(h.astype(bf), w2_ref[0].astype(bf),
                preferred_element_type=jnp.float32)
    h = jnp.maximum(h + b2_ref[0], 0.0)
    y = jnp.dot(h.astype(bf), w3_ref[0].astype(bf),
                preferred_element_type=jnp.float32)
    out_ref[...] = y + b3_ref[0]


def _run_mlp(xs, blk_e, w1, b1, w2, b2, w3, b3):
    grid_spec = pltpu.PrefetchScalarGridSpec(
        num_scalar_prefetch=1,
        grid=(NBLK,),
        in_specs=[
            pl.BlockSpec((BLK, D), lambda i, be: (i, 0)),
            pl.BlockSpec((1, D, H1), lambda i, be: (be[i], 0, 0)),
            pl.BlockSpec((1, 1, H1), lambda i, be: (be[i], 0, 0)),
            pl.BlockSpec((1, H1, H2), lambda i, be: (be[i], 0, 0)),
            pl.BlockSpec((1, 1, H2), lambda i, be: (be[i], 0, 0)),
            pl.BlockSpec((1, H2, DO), lambda i, be: (be[i], 0, 0)),
            pl.BlockSpec((1, 1, DO), lambda i, be: (be[i], 0, 0)),
        ],
        out_specs=pl.BlockSpec((BLK, DO), lambda i, be: (i, 0)),
    )
    return pl.pallas_call(
        _mlp_body,
        grid_spec=grid_spec,
        out_shape=jax.ShapeDtypeStruct((MAXPAD, DO), jnp.float32),
    )(blk_e, xs,
      w1, b1.reshape(E, 1, H1), w2, b2.reshape(E, 1, H2),
      w3, b3.reshape(E, 1, DO))


def _sc_combine(ys, pos0, pos1, g0rep, g1rep):
    tok_per_w = N // NW
    ch = 16
    mesh = plsc.VectorSubcoreMesh(core_axis_name="c", subcore_axis_name="s",
                                  num_cores=NC, num_subcores=NS)

    @functools.partial(
        pl.kernel, mesh=mesh,
        out_type=jax.ShapeDtypeStruct((N, DO), jnp.float32),
        scratch_types=[
            pltpu.VMEM((tok_per_w,), jnp.int32),
            pltpu.VMEM((tok_per_w,), jnp.int32),
            pltpu.VMEM((tok_per_w, 16), jnp.float32),
            pltpu.VMEM((tok_per_w, 16), jnp.float32),
            pltpu.VMEM((ch, DO), jnp.float32),
            pltpu.VMEM((ch, DO), jnp.float32),
            pltpu.VMEM((ch, DO), jnp.float32),
            pltpu.VMEM((ch, DO), jnp.float32),
            pltpu.SemaphoreType.DMA,
            pltpu.SemaphoreType.DMA,
            pltpu.SemaphoreType.DMA,
            pltpu.SemaphoreType.DMA,
            pltpu.SemaphoreType.DMA,
            pltpu.SemaphoreType.DMA,
        ],
    )
    def k(ys_hbm, p0_hbm, p1_hbm, g0_hbm, g1_hbm, out_hbm, i0_v, i1_v,
          g0_v, g1_v, a0, a1, b0, b1, ga0, ga1, gb0, gb1, s0, s1):
        wid = lax.axis_index("s") * NC + lax.axis_index("c")
        base = wid * tok_per_w
        pltpu.sync_copy(p0_hbm.at[pl.ds(base, tok_per_w)], i0_v)
        pltpu.sync_copy(p1_hbm.at[pl.ds(base, tok_per_w)], i1_v)
        pltpu.sync_copy(g0_hbm.at[pl.ds(base, tok_per_w)], g0_v)
        pltpu.sync_copy(g1_hbm.at[pl.ds(base, tok_per_w)], g1_v)
        abufs, bbufs = (a0, a1), (b0, b1)
        gasems, gbsems, ssems = (ga0, ga1), (gb0, gb1), (s0, s1)
        nch = tok_per_w // ch
        d0 = pltpu.async_copy(ys_hbm.at[i0_v.at[pl.ds(0, ch)]], a0, ga0)
        d1 = pltpu.async_copy(ys_hbm.at[i1_v.at[pl.ds(0, ch)]], b0, gb0)
        stores = [None, None]
        for c in range(nch):
            b = c & 1
            nd0 = nd1 = None
            if c + 1 < nch:
                ob = (c + 1) & 1
                if stores[ob] is not None:
                    stores[ob].wait()
                sl = pl.ds((c + 1) * ch, ch)
                nd0 = pltpu.async_copy(ys_hbm.at[i0_v.at[sl]], abufs[ob],
                                       gasems[ob])
                nd1 = pltpu.async_copy(ys_hbm.at[i1_v.at[sl]], bbufs[ob],
                                       gbsems[ob])
            d0.wait()
            d1.wait()
            a_ref, b_ref = abufs[b], bbufs[b]

            def row_body(r, _, a_ref=a_ref, b_ref=b_ref, c=c):
                ga = g0_v[c * ch + r, :]
                gb = g1_v[c * ch + r, :]
                for cc in range(DO // 16):
                    s = pl.ds(cc * 16, 16)
                    a_ref[r, s] = a_ref[r, s] * ga + b_ref[r, s] * gb
                return 0

            lax.fori_loop(0, ch, row_body, 0)
            stores[b] = pltpu.async_copy(a_ref,
                                         out_hbm.at[pl.ds(base + c * ch, ch)],
                                         ssems[b])
            d0, d1 = nd0, nd1
        stores[0].wait()
        stores[1].wait()

    return k(ys, pos0, pos1, g0rep, g1rep)


def kernel(inputs, Wr, br, expert_embeddings, W1, b1, W2, b2, W3, b3):
    i0, i1, g0, g1, r0, r1, cnt = _run_router(inputs, Wr, br,
                                              expert_embeddings)
    g0rep, g1rep, blk_e, pad_start_b = _route_metadata(g0, g1, cnt)
    xs, pos0, pos1 = _sc_scatter(inputs, i0.reshape(N), i1.reshape(N),
                                 r0.reshape(N), r1.reshape(N), pad_start_b)
    ys = _run_mlp(xs, blk_e, W1, b1, W2, b2, W3, b3)
    return _sc_combine(ys, pos0, pos1, g0rep, g1rep)
